# baseline (speedup 1.0000x reference)
import functools

import jax
import jax.numpy as jnp
from jax import lax
from jax.experimental import pallas as pl
from jax.experimental.pallas import tpu as pltpu
from jax.experimental.pallas import tpu_sc as plsc

OUT_H = 7
OUT_W = 7
P = OUT_H * OUT_W
SPATIAL_SCALE = 0.125
N, C, H, W = 2, 256, 128, 128
K = 1000
CHUNK = 104
QCH = 56
HA = 24
NW = 32
LANES = 16


def _index_body(rois_ref, idx_ref, wts_ref):
    r = rois_ref[...]
    b = r[:, 0:1].astype(jnp.int32)
    cx = r[:, 1:2] * SPATIAL_SCALE
    cy = r[:, 2:3] * SPATIAL_SCALE
    w = r[:, 3:4] * SPATIAL_SCALE
    h = r[:, 4:5] * SPATIAL_SCALE
    th = r[:, 5:6] * SPATIAL_SCALE
    cos_t = jnp.cos(th)
    sin_t = jnp.sin(th)
    p = lax.broadcasted_iota(jnp.int32, (1, P), 1)
    gy = ((p // OUT_W).astype(jnp.float32) + 0.5) / OUT_H - 0.5
    gx = ((p % OUT_W).astype(jnp.float32) + 0.5) / OUT_W - 0.5
    gxw = gx * w
    gyh = gy * h
    ix = gxw * cos_t - gyh * sin_t + cx - 0.5
    iy = gxw * sin_t + gyh * cos_t + cy - 0.5
    x0 = jnp.floor(ix)
    y0 = jnp.floor(iy)
    x1 = x0 + 1.0
    y1 = y0 + 1.0
    wx1 = ix - x0
    wx0 = 1.0 - wx1
    wy1 = iy - y0
    wy0 = 1.0 - wy1
    base = b * (H * W)

    xs = jnp.clip(x0, 0, W - 2)
    ys = jnp.clip(y0, 0, H - 2)
    xsi = xs.astype(jnp.int32)
    ysi = ys.astype(jnp.int32)
    wxs0 = wx0 * (xs == x0).astype(jnp.float32) + wx1 * (xs == x1).astype(jnp.float32)
    xsp = xs + 1.0
    wxs1 = wx0 * (xsp == x0).astype(jnp.float32) + wx1 * (xsp == x1).astype(jnp.float32)
    wys0 = wy0 * (ys == y0).astype(jnp.float32) + wy1 * (ys == y1).astype(jnp.float32)
    ysp = ys + 1.0
    wys1 = wy0 * (ysp == y0).astype(jnp.float32) + wy1 * (ysp == y1).astype(jnp.float32)
    iq = base + ysi * W + xsi
    zi = jnp.zeros((K, QCH - P), jnp.int32)
    zf = jnp.zeros((K, CHUNK - 2 * P), jnp.float32)
    idx_ref[...] = jnp.concatenate([iq, zi], axis=1)
    wts_ref[...] = jnp.concatenate(
        [wys0 * wxs0, wys0 * wxs1, zf,
         wys1 * wxs0, wys1 * wxs1, zf], axis=1)


def _build_indices(rois):
    return pl.pallas_call(
        _index_body,
        out_shape=[
            jax.ShapeDtypeStruct((K, QCH), jnp.int32),
            jax.ShapeDtypeStruct((K, 2 * CHUNK), jnp.float32),
        ],
    )(rois)


def _sc_gather_fn():
    mesh = plsc.VectorSubcoreMesh(core_axis_name="c", subcore_axis_name="s")
    WROI = 4 * P * LANES
    TCH = 32 * QCH
    HB = QCH - HA

    @functools.partial(
        pl.kernel,
        mesh=mesh,
        out_type=jax.ShapeDtypeStruct((K * P * C,), jnp.float32),
        scratch_types=[
            pltpu.VMEM((TCH,), jnp.int32),
            pltpu.VMEM((WROI,), jnp.float32),
            pltpu.VMEM((WROI,), jnp.float32),
            pltpu.VMEM((HA, 4 * C), jnp.float32),
            pltpu.VMEM((HB, 4 * C), jnp.float32),
            pltpu.VMEM((P * C,), jnp.float32),
            pltpu.SemaphoreType.DMA,
            pltpu.SemaphoreType.DMA,
            pltpu.SemaphoreType.DMA,
            pltpu.SemaphoreType.DMA,
            pltpu.SemaphoreType.DMA,
        ],
    )
    def sc_gather(quad_hbm, idxp_hbm, wts_hbm, out_hbm,
                  idx_all, wv0, wv1, bufa, bufb, outb,
                  sga, sgb, sw0, sw1, so):
        wv = (wv0, wv1)
        buf = (bufa, bufb)
        sg = (sga, sgb)
        sw = (sw0, sw1)
        wid = lax.axis_index("s") * 2 + lax.axis_index("c")
        nk = (K - 1 - wid) // NW + 1

        def gather_desc(h, t):
            off = t * QCH + (0 if h == 0 else HA)
            n = HA if h == 0 else HB
            return pltpu.make_async_copy(
                quad_hbm.at[idx_all.at[pl.ds(off, n)]], buf[h], sg[h])

        def wv_desc(t, w):
            k = wid + t * NW
            return pltpu.make_async_copy(
                wts_hbm.at[pl.ds(k * WROI, WROI)], wv[w], sw[w])

        def out_slice(t):
            k = wid + t * NW
            return out_hbm.at[pl.ds(k * P * C, P * C)]

        def compute_half(h, w):
            bb, w_v = buf[h], wv[w]
            lo, hi, base = (0, HA, 0) if h == 0 else (HA, P, HA)

            @plsc.parallel_loop(lo, hi, unroll=4)
            def point_body(pp):
                row = pp - base
                w0 = w_v[pl.ds(pp * LANES, LANES)]
                w1 = w_v[pl.ds((P + pp) * LANES, LANES)]
                w2 = w_v[pl.ds((2 * P + pp) * LANES, LANES)]
                w3 = w_v[pl.ds((3 * P + pp) * LANES, LANES)]
                for c0 in range(C // LANES):
                    o = c0 * LANES
                    acc = (bb[row, pl.ds(o, LANES)] * w0
                           + bb[row, pl.ds(C + o, LANES)] * w1
                           + bb[row, pl.ds(2 * C + o, LANES)] * w2
                           + bb[row, pl.ds(3 * C + o, LANES)] * w3)
                    outb[pl.ds(pp * C + o, LANES)] = acc

        pltpu.sync_copy(idxp_hbm.at[pl.ds(wid * TCH, TCH)], idx_all)
        gather_desc(0, 0).start()
        wv_desc(0, 0).start()
        gather_desc(1, 0).start()
        n2 = (nk + 1) // 2

        def outer(i2, _):
            for w in (0, 1):
                t = i2 * 2 + w

                @pl.when(t < nk)
                def _():
                    gather_desc(0, t).wait()
                    wv_desc(t, w).wait()

                    @pl.when(t >= 1)
                    def _():
                        pltpu.make_async_copy(outb, out_slice(t - 1), so).wait()

                    compute_half(0, w)

                    @pl.when(t + 1 < nk)
                    def _():
                        gather_desc(0, t + 1).start()
                        wv_desc(t + 1, 1 - w).start()

                    gather_desc(1, t).wait()
                    compute_half(1, w)

                    @pl.when(t + 1 < nk)
                    def _():
                        gather_desc(1, t + 1).start()

                    pltpu.async_copy(outb, out_slice(t), so)
            return 0

        lax.fori_loop(0, n2, outer, 0)
        pltpu.make_async_copy(outb, out_slice(nk - 1), so).wait()

    return sc_gather


_SC_GATHER = _sc_gather_fn()


def kernel(features, rois):
    feats_flat = features.transpose(0, 2, 3, 1).reshape(N * H * W, C)
    ffp = jnp.pad(feats_flat, ((0, 129), (0, 0)))
    quad = jnp.concatenate(
        [ffp[:-129], ffp[1:-128], ffp[128:-1], ffp[129:]], axis=1)
    idx, wts = _build_indices(rois)
    w4 = jnp.concatenate([wts[:, :2 * P], wts[:, CHUNK:CHUNK + 2 * P]], axis=1)
    w16 = jnp.broadcast_to(w4[:, :, None], (K, 4 * P, LANES)).reshape(-1)
    idxp = jnp.pad(idx, ((0, 32 * NW - K), (0, 0)))
    idx_perm = idxp.reshape(32, NW, QCH).transpose(1, 0, 2).reshape(-1)
    rows = _SC_GATHER(quad, idx_perm, w16)
    return rows.reshape(K, P, C).transpose(0, 2, 1).reshape(K, C, OUT_H, OUT_W)


# speedup vs baseline: 1.5368x; 1.5368x over previous
import functools

import jax
import jax.numpy as jnp
from jax import lax
from jax.experimental import pallas as pl
from jax.experimental.pallas import tpu as pltpu
from jax.experimental.pallas import tpu_sc as plsc

OUT_H = 7
OUT_W = 7
P = OUT_H * OUT_W
SPATIAL_SCALE = 0.125
N, C, H, W = 2, 256, 128, 128
K = 1000
CHUNK = 104
QCH = 56
HA = 24
NW = 32
LANES = 16


def _index_body(rois_ref, idx_ref, wts_ref):
    r = rois_ref[...]
    b = r[:, 0:1].astype(jnp.int32)
    cx = r[:, 1:2] * SPATIAL_SCALE
    cy = r[:, 2:3] * SPATIAL_SCALE
    w = r[:, 3:4] * SPATIAL_SCALE
    h = r[:, 4:5] * SPATIAL_SCALE
    th = r[:, 5:6] * SPATIAL_SCALE
    cos_t = jnp.cos(th)
    sin_t = jnp.sin(th)
    p = lax.broadcasted_iota(jnp.int32, (1, P), 1)
    gy = ((p // OUT_W).astype(jnp.float32) + 0.5) / OUT_H - 0.5
    gx = ((p % OUT_W).astype(jnp.float32) + 0.5) / OUT_W - 0.5
    gxw = gx * w
    gyh = gy * h
    ix = gxw * cos_t - gyh * sin_t + cx - 0.5
    iy = gxw * sin_t + gyh * cos_t + cy - 0.5
    x0 = jnp.floor(ix)
    y0 = jnp.floor(iy)
    x1 = x0 + 1.0
    y1 = y0 + 1.0
    wx1 = ix - x0
    wx0 = 1.0 - wx1
    wy1 = iy - y0
    wy0 = 1.0 - wy1
    base = b * (H * W)

    xs = jnp.clip(x0, 0, W - 2)
    ys = jnp.clip(y0, 0, H - 2)
    xsi = xs.astype(jnp.int32)
    ysi = ys.astype(jnp.int32)
    wxs0 = wx0 * (xs == x0).astype(jnp.float32) + wx1 * (xs == x1).astype(jnp.float32)
    xsp = xs + 1.0
    wxs1 = wx0 * (xsp == x0).astype(jnp.float32) + wx1 * (xsp == x1).astype(jnp.float32)
    wys0 = wy0 * (ys == y0).astype(jnp.float32) + wy1 * (ys == y1).astype(jnp.float32)
    ysp = ys + 1.0
    wys1 = wy0 * (ysp == y0).astype(jnp.float32) + wy1 * (ysp == y1).astype(jnp.float32)
    iq = base + ysi * W + xsi
    zi = jnp.zeros((K, QCH - P), jnp.int32)
    zf = jnp.zeros((K, CHUNK - 2 * P), jnp.float32)
    idx_ref[...] = jnp.concatenate([iq, zi], axis=1)
    wts_ref[...] = jnp.concatenate(
        [wys0 * wxs0, wys0 * wxs1, zf,
         wys1 * wxs0, wys1 * wxs1, zf], axis=1)


def _build_indices(rois):
    return pl.pallas_call(
        _index_body,
        out_shape=[
            jax.ShapeDtypeStruct((K, QCH), jnp.int32),
            jax.ShapeDtypeStruct((K, 2 * CHUNK), jnp.float32),
        ],
    )(rois)


def _sc_gather_fn():
    mesh = plsc.VectorSubcoreMesh(core_axis_name="c", subcore_axis_name="s")
    WROI = 4 * P * LANES
    TCH = 32 * QCH

    @functools.partial(
        pl.kernel,
        mesh=mesh,
        out_type=jax.ShapeDtypeStruct((K, P, C), jnp.float32),
        scratch_types=[
            pltpu.VMEM((TCH,), jnp.int32),
            pltpu.VMEM((WROI,), jnp.float32),
            pltpu.VMEM((WROI,), jnp.float32),
            pltpu.VMEM((P, 4 * C), jnp.float32),
            pltpu.VMEM((P, 4 * C), jnp.float32),
            pltpu.SemaphoreType.DMA,
            pltpu.SemaphoreType.DMA,
            pltpu.SemaphoreType.DMA,
            pltpu.SemaphoreType.DMA,
            pltpu.SemaphoreType.DMA,
            pltpu.SemaphoreType.DMA,
        ],
    )
    def sc_gather(quad_hbm, idxp_hbm, wts_hbm, out_hbm,
                  idx_all, wv0, wv1, buf0, buf1,
                  sg0, sg1, sw0, sw1, so0, so1):
        wv = (wv0, wv1)
        buf = (buf0, buf1)
        sg = (sg0, sg1)
        sw = (sw0, sw1)
        so = (so0, so1)
        wid = lax.axis_index("s") * 2 + lax.axis_index("c")
        nk = (K - 1 - wid) // NW + 1

        def gather_desc(b, t):
            return pltpu.make_async_copy(
                quad_hbm.at[idx_all.at[pl.ds(t * QCH, P)]], buf[b], sg[b])

        def wv_desc(b, t):
            k = wid + t * NW
            return pltpu.make_async_copy(
                wts_hbm.at[pl.ds(k * WROI, WROI)], wv[b], sw[b])

        def out_desc(b, t):
            k = wid + t * NW
            return pltpu.make_async_copy(
                buf[b].at[:, pl.ds(0, C)], out_hbm.at[k], so[b])

        def stage(b, t):
            gather_desc(b, t).start()
            wv_desc(b, t).start()

        def compute(b):
            bb, w_v = buf[b], wv[b]

            @plsc.parallel_loop(0, P, unroll=4)
            def point_body(pp):
                w0 = w_v[pl.ds(pp * LANES, LANES)]
                w1 = w_v[pl.ds((P + pp) * LANES, LANES)]
                w2 = w_v[pl.ds((2 * P + pp) * LANES, LANES)]
                w3 = w_v[pl.ds((3 * P + pp) * LANES, LANES)]
                for c0 in range(C // LANES):
                    o = c0 * LANES
                    acc = (bb[pp, pl.ds(o, LANES)] * w0
                           + bb[pp, pl.ds(C + o, LANES)] * w1
                           + bb[pp, pl.ds(2 * C + o, LANES)] * w2
                           + bb[pp, pl.ds(3 * C + o, LANES)] * w3)
                    bb[pp, pl.ds(o, LANES)] = acc

        pltpu.sync_copy(idxp_hbm.at[pl.ds(wid * TCH, TCH)], idx_all)
        stage(0, 0)
        stage(1, 1)
        n2 = (nk + 1) // 2

        def outer(i2, _):
            for b in (0, 1):
                t = i2 * 2 + b

                @pl.when(t < nk)
                def _():
                    gather_desc(b, t).wait()
                    wv_desc(b, t).wait()
                    compute(b)

                    @pl.when(t >= 1)
                    def _():
                        out_desc(1 - b, t - 1).wait()

                        @pl.when(t + 1 < nk)
                        def _():
                            stage(1 - b, t + 1)

                    out_desc(b, t).start()
            return 0

        lax.fori_loop(0, n2, outer, 0)

        @pl.when(nk % 2 == 1)
        def _():
            out_desc(0, 0).wait()

        @pl.when(nk % 2 == 0)
        def _():
            out_desc(1, 0).wait()

    return sc_gather


_SC_GATHER = _sc_gather_fn()


def kernel(features, rois):
    feats_flat = features.transpose(0, 2, 3, 1).reshape(N * H * W, C)
    ffp = jnp.pad(feats_flat, ((0, 129), (0, 0)))
    quad = jnp.concatenate(
        [ffp[:-129], ffp[1:-128], ffp[128:-1], ffp[129:]], axis=1)
    idx, wts = _build_indices(rois)
    w4 = jnp.concatenate([wts[:, :2 * P], wts[:, CHUNK:CHUNK + 2 * P]], axis=1)
    w16 = jnp.broadcast_to(w4[:, :, None], (K, 4 * P, LANES)).reshape(-1)
    idxp = jnp.pad(idx, ((0, 32 * NW - K), (0, 0)))
    idx_perm = idxp.reshape(32, NW, QCH).transpose(1, 0, 2).reshape(-1)
    rows = _SC_GATHER(quad, idx_perm, w16)
    return rows.transpose(0, 2, 1).reshape(K, C, OUT_H, OUT_W)
